# Initial kernel scaffold; baseline (speedup 1.0000x reference)
import jax
import jax.numpy as jnp
from jax.experimental import pallas as pl


def kernel(x, edge_index, Wl1, Wr1, b1, g1, be1, Wl2, Wr2, b2, g2, be2, Wl3, Wr3, b3, g3, be3):
    raise NotImplementedError("write your pallas kernel here")


# speedup vs baseline: 4.2883x; 4.2883x over previous
import functools

import jax
import jax.numpy as jnp
from jax import lax
from jax.experimental import pallas as pl
from jax.experimental.pallas import tpu as pltpu
from jax.experimental.pallas import tpu_sc as plsc

NC = 2
NS = 16
NW = NC * NS
CHUNK = 128
BLK = 400


def _ln(z, g, b, eps=1e-5):
    mu = jnp.mean(z, axis=1, keepdims=True)
    d = z - mu
    var = jnp.mean(d * d, axis=1, keepdims=True)
    return d * lax.rsqrt(var + eps) * g + b


def _dot_t(a, w):
    return lax.dot_general(a, w, (((1,), (1,)), ((), ())),
                           preferred_element_type=jnp.float32)


def _make_agg(d, np_rows, c_chunks):
    r = np_rows // NS
    mesh = plsc.VectorSubcoreMesh(core_axis_name="c", subcore_axis_name="s")

    @functools.partial(
        pl.kernel,
        mesh=mesh,
        out_type=jax.ShapeDtypeStruct((NC, np_rows, d), jnp.float32),
        scratch_types=[
            pltpu.VMEM((c_chunks, CHUNK), jnp.int32),
            pltpu.VMEM((c_chunks, CHUNK), jnp.int32),
            pltpu.VMEM((CHUNK, d), jnp.float32),
            pltpu.VMEM_SHARED((np_rows, d), jnp.float32),
            pltpu.SemaphoreType.DMA,
        ],
        compiler_params=pltpu.CompilerParams(use_tc_tiling_on_sc=False),
    )
    def agg(y_hbm, srcb_hbm, dstb_hbm, zeros_hbm, out_hbm,
            src_v, dst_v, rows_v, acc, sem):
        cid = lax.axis_index("c")
        sid = lax.axis_index("s")
        wid = sid * NC + cid
        pltpu.sync_copy(srcb_hbm.at[wid], src_v)
        pltpu.sync_copy(dstb_hbm.at[wid], dst_v)
        r0 = sid * r
        pltpu.sync_copy(zeros_hbm.at[pl.ds(r0, r)], acc.at[pl.ds(r0, r)])
        plsc.subcore_barrier()

        def body(j, carry):
            pltpu.async_copy(y_hbm.at[src_v.at[j]], rows_v, sem).wait()
            pltpu.sync_copy(rows_v, acc.at[dst_v.at[j]], add=True)
            return carry

        lax.fori_loop(0, c_chunks, body, 0)
        plsc.subcore_barrier()
        pltpu.sync_copy(acc.at[pl.ds(r0, r)], out_hbm.at[cid, pl.ds(r0, r)])

    return agg


def _pre(x, wl1):
    n = x.shape[0]
    grid = (n // BLK,)

    def body(x_ref, w_ref, o_ref):
        y = _dot_t(x_ref[...], w_ref[...])
        ones = jnp.ones((BLK, 16), jnp.float32)
        o_ref[...] = jnp.concatenate([y, ones], axis=1)

    return pl.pallas_call(
        body,
        grid=grid,
        in_specs=[
            pl.BlockSpec((BLK, x.shape[1]), lambda i: (i, 0)),
            pl.BlockSpec(wl1.shape, lambda i: (0, 0)),
        ],
        out_specs=pl.BlockSpec((BLK, wl1.shape[0] + 16), lambda i: (i, 0)),
        out_shape=jax.ShapeDtypeStruct((n, wl1.shape[0] + 16), jnp.float32),
    )(x, wl1)


def _mid(h_in, p, rinv, wr, b, g, be, wl_next):
    n = h_in.shape[0]
    d_out = wr.shape[0]
    np_rows = p.shape[1]
    d_agg = p.shape[2]
    first = rinv is None
    grid = (n // BLK,)

    def body2(*refs):
        i = 0
        h_ref = refs[i]; i += 1
        p_ref = refs[i]; i += 1
        if not first:
            rinv_ref = refs[i]; i += 1
        wr_ref = refs[i]; i += 1
        b_ref = refs[i]; i += 1
        g_ref = refs[i]; i += 1
        be_ref = refs[i]; i += 1
        if wl_next is not None:
            wln_ref = refs[i]; i += 1
        h_out = refs[i]; i += 1
        if wl_next is not None:
            y_out = refs[i]; i += 1
        if first:
            rinv_out = refs[i]; i += 1

        s = p_ref[0] + p_ref[1]
        if first:
            cnt = s[:, d_out:d_out + 1]
            ri = 1.0 / jnp.maximum(cnt, 1.0)
        else:
            ri = rinv_ref[:, 0:1]
        mean = s[:, :d_out] * ri
        z = mean + b_ref[...] + _dot_t(h_ref[...], wr_ref[...])
        h = _ln(jnp.maximum(z, 0.0), g_ref[...], be_ref[...])
        h_out[...] = h
        if wl_next is not None:
            y_out[...] = _dot_t(h, wln_ref[...])
        if first:
            rinv_out[...] = jnp.broadcast_to(ri, (BLK, 16))

    in_specs = [
        pl.BlockSpec((BLK, h_in.shape[1]), lambda i: (i, 0)),
        pl.BlockSpec((NC, BLK, d_agg), lambda i: (0, i, 0)),
    ]
    operands = [h_in, p]
    if not first:
        in_specs.append(pl.BlockSpec((BLK, 16), lambda i: (i, 0)))
        operands.append(rinv)
    in_specs += [
        pl.BlockSpec(wr.shape, lambda i: (0, 0)),
        pl.BlockSpec((1, d_out), lambda i: (0, 0)),
        pl.BlockSpec((1, d_out), lambda i: (0, 0)),
        pl.BlockSpec((1, d_out), lambda i: (0, 0)),
    ]
    operands += [wr, b.reshape(1, -1), g.reshape(1, -1), be.reshape(1, -1)]
    if wl_next is not None:
        in_specs.append(pl.BlockSpec(wl_next.shape, lambda i: (0, 0)))
        operands.append(wl_next)

    out_specs = [pl.BlockSpec((BLK, d_out), lambda i: (i, 0))]
    out_shape = [jax.ShapeDtypeStruct((n, d_out), jnp.float32)]
    if wl_next is not None:
        out_specs.append(pl.BlockSpec((BLK, wl_next.shape[0]), lambda i: (i, 0)))
        out_shape.append(jax.ShapeDtypeStruct((n, wl_next.shape[0]), jnp.float32))
    if first:
        out_specs.append(pl.BlockSpec((BLK, 16), lambda i: (i, 0)))
        out_shape.append(jax.ShapeDtypeStruct((n, 16), jnp.float32))

    return pl.pallas_call(
        body2,
        grid=grid,
        in_specs=in_specs,
        out_specs=out_specs,
        out_shape=out_shape,
    )(*operands)


def kernel(x, edge_index, Wl1, Wr1, b1, g1, be1,
           Wl2, Wr2, b2, g2, be2, Wl3, Wr3, b3, g3, be3):
    n = x.shape[0]
    e = edge_index.shape[1]
    c_chunks = -(-e // (NW * CHUNK))
    e_pad = NW * c_chunks * CHUNK
    r = -(-(n + 1) // (NS * 8)) * 8
    np_rows = NS * r

    ei = edge_index.astype(jnp.int32)
    src = jnp.concatenate([ei[0], jnp.zeros((e_pad - e,), jnp.int32)])
    dst = jnp.concatenate([ei[1], jnp.full((e_pad - e,), n, jnp.int32)])
    srcb = src.reshape(NW, c_chunks, CHUNK)
    dstb = dst.reshape(NW, c_chunks, CHUNK)

    d1 = Wl1.shape[0] + 16
    d23 = Wl2.shape[0]
    z1 = jnp.zeros((np_rows, d1), jnp.float32)
    z23 = jnp.zeros((np_rows, d23), jnp.float32)

    agg1 = _make_agg(d1, np_rows, c_chunks)
    agg23 = _make_agg(d23, np_rows, c_chunks)

    y1 = _pre(x, Wl1)
    p1 = agg1(y1, srcb, dstb, z1)
    h1, y2, rinv = _mid(x, p1, None, Wr1, b1, g1, be1, Wl2)
    p2 = agg23(y2, srcb, dstb, z23)
    h2, y3 = _mid(h1, p2, rinv, Wr2, b2, g2, be2, Wl3)
    p3 = agg23(y3, srcb, dstb, z23)
    (h3,) = _mid(h2, p3, rinv, Wr3, b3, g3, be3, None)
    return h3


# speedup vs baseline: 4.4834x; 1.0455x over previous
import functools

import jax
import jax.numpy as jnp
from jax import lax
from jax.experimental import pallas as pl
from jax.experimental.pallas import tpu as pltpu
from jax.experimental.pallas import tpu_sc as plsc

NC = 2
NS = 16
NW = NC * NS
CHUNK = 128
BLK = 400


def _ln(z, g, b, eps=1e-5):
    mu = jnp.mean(z, axis=1, keepdims=True)
    d = z - mu
    var = jnp.mean(d * d, axis=1, keepdims=True)
    return d * lax.rsqrt(var + eps) * g + b


def _dot_t(a, w):
    return lax.dot_general(a, w, (((1,), (1,)), ((), ())),
                           preferred_element_type=jnp.float32)


def _make_agg(d, np_rows, c_chunks):
    r = np_rows // NS
    mesh = plsc.VectorSubcoreMesh(core_axis_name="c", subcore_axis_name="s")

    @functools.partial(
        pl.kernel,
        mesh=mesh,
        out_type=jax.ShapeDtypeStruct((NC, np_rows, d), jnp.float32),
        scratch_types=[
            pltpu.VMEM((c_chunks, CHUNK), jnp.int32),
            pltpu.VMEM((c_chunks, CHUNK), jnp.int32),
            pltpu.VMEM((CHUNK, d), jnp.float32),
            pltpu.VMEM((CHUNK, d), jnp.float32),
            pltpu.VMEM_SHARED((np_rows, d), jnp.float32),
            pltpu.SemaphoreType.DMA,
            pltpu.SemaphoreType.DMA,
        ],
        compiler_params=pltpu.CompilerParams(use_tc_tiling_on_sc=False),
    )
    def agg(y_hbm, srcb_hbm, dstb_hbm, zeros_hbm, out_hbm,
            src_v, dst_v, buf0, buf1, acc, sem0, sem1):
        cid = lax.axis_index("c")
        sid = lax.axis_index("s")
        wid = sid * NC + cid
        pltpu.sync_copy(srcb_hbm.at[wid], src_v)
        pltpu.sync_copy(dstb_hbm.at[wid], dst_v)
        r0 = sid * r
        pltpu.sync_copy(zeros_hbm.at[pl.ds(r0, r)], acc.at[pl.ds(r0, r)])
        plsc.subcore_barrier()

        last = c_chunks - 1
        pltpu.async_copy(y_hbm.at[src_v.at[0]], buf0, sem0)
        pltpu.async_copy(y_hbm.at[src_v.at[jnp.minimum(1, last)]], buf1, sem1)

        def body(i, carry):
            j = 2 * i
            pltpu.make_async_copy(y_hbm.at[src_v.at[j]], buf0, sem0).wait()
            pltpu.sync_copy(buf0, acc.at[dst_v.at[j]], add=True)
            pltpu.async_copy(
                y_hbm.at[src_v.at[jnp.minimum(j + 2, last)]], buf0, sem0)
            pltpu.make_async_copy(y_hbm.at[src_v.at[j]], buf1, sem1).wait()
            pltpu.sync_copy(buf1, acc.at[dst_v.at[jnp.minimum(j + 1, last)]],
                            add=True)
            pltpu.async_copy(
                y_hbm.at[src_v.at[jnp.minimum(j + 3, last)]], buf1, sem1)
            return carry

        lax.fori_loop(0, (c_chunks + 1) // 2, body, 0)
        pltpu.make_async_copy(y_hbm.at[src_v.at[0]], buf0, sem0).wait()
        pltpu.make_async_copy(y_hbm.at[src_v.at[0]], buf1, sem1).wait()
        plsc.subcore_barrier()
        pltpu.sync_copy(acc.at[pl.ds(r0, r)], out_hbm.at[cid, pl.ds(r0, r)])

    return agg


def _make_cnt(np_rows, c_chunks):
    r = np_rows // NS
    mesh = plsc.VectorSubcoreMesh(core_axis_name="c", subcore_axis_name="s")

    @functools.partial(
        pl.kernel,
        mesh=mesh,
        out_type=jax.ShapeDtypeStruct((NC, np_rows, 16), jnp.float32),
        scratch_types=[
            pltpu.VMEM((c_chunks, CHUNK), jnp.int32),
            pltpu.VMEM((CHUNK, 16), jnp.float32),
            pltpu.VMEM_SHARED((np_rows, 16), jnp.float32),
        ],
        compiler_params=pltpu.CompilerParams(use_tc_tiling_on_sc=False),
    )
    def cnt(dstb_hbm, zeros_hbm, ones_hbm, out_hbm, dst_v, ones_v, acc):
        cid = lax.axis_index("c")
        sid = lax.axis_index("s")
        wid = sid * NC + cid
        pltpu.sync_copy(dstb_hbm.at[wid], dst_v)
        pltpu.sync_copy(ones_hbm, ones_v)
        r0 = sid * r
        pltpu.sync_copy(zeros_hbm.at[pl.ds(r0, r)], acc.at[pl.ds(r0, r)])
        plsc.subcore_barrier()

        def body(j, carry):
            pltpu.sync_copy(ones_v, acc.at[dst_v.at[j]], add=True)
            return carry

        lax.fori_loop(0, c_chunks, body, 0)
        plsc.subcore_barrier()
        pltpu.sync_copy(acc.at[pl.ds(r0, r)], out_hbm.at[cid, pl.ds(r0, r)])

    return cnt


def _pre(x, wl1):
    n = x.shape[0]
    grid = (n // BLK,)

    def body(x_ref, w_ref, o_ref):
        o_ref[...] = _dot_t(x_ref[...], w_ref[...])

    return pl.pallas_call(
        body,
        grid=grid,
        in_specs=[
            pl.BlockSpec((BLK, x.shape[1]), lambda i: (i, 0)),
            pl.BlockSpec(wl1.shape, lambda i: (0, 0)),
        ],
        out_specs=pl.BlockSpec((BLK, wl1.shape[0]), lambda i: (i, 0)),
        out_shape=jax.ShapeDtypeStruct((n, wl1.shape[0]), jnp.float32),
    )(x, wl1)


def _mid(h_in, p, cr, wr, b, g, be, wl_next, first):
    n = h_in.shape[0]
    d_out = wr.shape[0]
    np_rows = p.shape[1]
    d_agg = p.shape[2]
    grid = (n // BLK,)

    def body2(*refs):
        i = 0
        h_ref = refs[i]; i += 1
        p_ref = refs[i]; i += 1
        cr_ref = refs[i]; i += 1
        wr_ref = refs[i]; i += 1
        b_ref = refs[i]; i += 1
        g_ref = refs[i]; i += 1
        be_ref = refs[i]; i += 1
        if wl_next is not None:
            wln_ref = refs[i]; i += 1
        h_out = refs[i]; i += 1
        if wl_next is not None:
            y_out = refs[i]; i += 1
        if first:
            rinv_out = refs[i]; i += 1

        s = p_ref[0] + p_ref[1]
        if first:
            cnt = cr_ref[0, :, 0:1] + cr_ref[1, :, 0:1]
            ri = 1.0 / jnp.maximum(cnt, 1.0)
        else:
            ri = cr_ref[:, 0:1]
        mean = s * ri
        z = mean + b_ref[...] + _dot_t(h_ref[...], wr_ref[...])
        h = _ln(jnp.maximum(z, 0.0), g_ref[...], be_ref[...])
        h_out[...] = h
        if wl_next is not None:
            y_out[...] = _dot_t(h, wln_ref[...])
        if first:
            rinv_out[...] = jnp.broadcast_to(ri, (BLK, 16))

    in_specs = [
        pl.BlockSpec((BLK, h_in.shape[1]), lambda i: (i, 0)),
        pl.BlockSpec((NC, BLK, d_agg), lambda i: (0, i, 0)),
    ]
    operands = [h_in, p]
    if first:
        in_specs.append(pl.BlockSpec((NC, BLK, 16), lambda i: (0, i, 0)))
    else:
        in_specs.append(pl.BlockSpec((BLK, 16), lambda i: (i, 0)))
    operands.append(cr)
    in_specs += [
        pl.BlockSpec(wr.shape, lambda i: (0, 0)),
        pl.BlockSpec((1, d_out), lambda i: (0, 0)),
        pl.BlockSpec((1, d_out), lambda i: (0, 0)),
        pl.BlockSpec((1, d_out), lambda i: (0, 0)),
    ]
    operands += [wr, b.reshape(1, -1), g.reshape(1, -1), be.reshape(1, -1)]
    if wl_next is not None:
        in_specs.append(pl.BlockSpec(wl_next.shape, lambda i: (0, 0)))
        operands.append(wl_next)

    out_specs = [pl.BlockSpec((BLK, d_out), lambda i: (i, 0))]
    out_shape = [jax.ShapeDtypeStruct((n, d_out), jnp.float32)]
    if wl_next is not None:
        out_specs.append(pl.BlockSpec((BLK, wl_next.shape[0]), lambda i: (i, 0)))
        out_shape.append(jax.ShapeDtypeStruct((n, wl_next.shape[0]), jnp.float32))
    if first:
        out_specs.append(pl.BlockSpec((BLK, 16), lambda i: (i, 0)))
        out_shape.append(jax.ShapeDtypeStruct((n, 16), jnp.float32))

    return pl.pallas_call(
        body2,
        grid=grid,
        in_specs=in_specs,
        out_specs=out_specs,
        out_shape=out_shape,
    )(*operands)


def kernel(x, edge_index, Wl1, Wr1, b1, g1, be1,
           Wl2, Wr2, b2, g2, be2, Wl3, Wr3, b3, g3, be3):
    n = x.shape[0]
    e = edge_index.shape[1]
    c_chunks = 2 * -(-e // (NW * CHUNK * 2))
    e_pad = NW * c_chunks * CHUNK
    r = -(-(n + 1) // (NS * 8)) * 8
    np_rows = NS * r

    ei = edge_index.astype(jnp.int32)
    src = jnp.concatenate([ei[0], jnp.zeros((e_pad - e,), jnp.int32)])
    dst = jnp.concatenate([ei[1], jnp.full((e_pad - e,), n, jnp.int32)])
    srcb = src.reshape(NW, c_chunks, CHUNK)
    dstb = dst.reshape(NW, c_chunks, CHUNK)

    d1 = Wl1.shape[0]
    d23 = Wl2.shape[0]
    z1 = jnp.zeros((np_rows, d1), jnp.float32)
    z23 = jnp.zeros((np_rows, d23), jnp.float32)
    z16 = jnp.zeros((np_rows, 16), jnp.float32)
    ones16 = jnp.ones((CHUNK, 16), jnp.float32)

    agg1 = _make_agg(d1, np_rows, c_chunks)
    agg23 = _make_agg(d23, np_rows, c_chunks)
    cntk = _make_cnt(np_rows, c_chunks)

    cntp = cntk(dstb, z16, ones16)
    y1 = _pre(x, Wl1)
    p1 = agg1(y1, srcb, dstb, z1)
    h1, y2, rinv = _mid(x, p1, cntp, Wr1, b1, g1, be1, Wl2, True)
    p2 = agg23(y2, srcb, dstb, z23)
    h2, y3 = _mid(h1, p2, rinv, Wr2, b2, g2, be2, Wl3, False)
    p3 = agg23(y3, srcb, dstb, z23)
    (h3,) = _mid(h2, p3, rinv, Wr3, b3, g3, be3, None, False)
    return h3
